# Initial kernel scaffold; baseline (speedup 1.0000x reference)
import jax
import jax.numpy as jnp
from jax.experimental import pallas as pl


def kernel(x, edge_index, batch, W1, b1, W2, b2, lin_W, lin_b):
    raise NotImplementedError("write your pallas kernel here")


# speedup vs baseline: 13.6753x; 13.6753x over previous
import functools

import jax
import jax.numpy as jnp
from jax import lax
from jax.experimental import pallas as pl
from jax.experimental.pallas import tpu as pltpu
from jax.experimental.pallas import tpu_sc as plsc

N = 10000
E = 320000
D = 128
G = 64
NC, NS = 2, 16
NW = NC * NS
NPAD = 10240
EPW = E // NW
K = 80
NCHUNK = EPW // K
RPT = NPAD // NS
CH = 128
R = 2048
DW = 16

_sc_mesh = plsc.VectorSubcoreMesh(
    core_axis_name="c", subcore_axis_name="s", num_cores=NC, num_subcores=NS)

_sc_params = pltpu.CompilerParams(
    use_tc_tiling_on_sc=False, needs_layout_passes=False)


@functools.partial(
    pl.kernel,
    out_type=jax.ShapeDtypeStruct((NW, NPAD), jnp.float32),
    mesh=_sc_mesh,
    compiler_params=_sc_params,
    scratch_types=[
        pltpu.VMEM((K,), jnp.int32),
        pltpu.VMEM((NPAD,), jnp.float32),
    ],
)
def _sc_degree(dst_hbm, out_hbm, didx, deg_v):
    c = lax.axis_index("c")
    s = lax.axis_index("s")
    w = c * NS + s

    @pl.loop(0, NPAD // 16)
    def _zero(i):
        deg_v[pl.ds(i * 16, 16)] = jnp.zeros((16,), jnp.float32)

    base = w * EPW
    ones16 = jnp.ones((16,), jnp.float32)

    @pl.loop(0, NCHUNK)
    def _chunk(i):
        pltpu.sync_copy(dst_hbm.at[pl.ds(base + i * K, K)], didx)
        for j in range(K // 16):
            idx = didx[pl.ds(j * 16, 16)]
            plsc.addupdate_scatter(deg_v, [idx], ones16)

    pltpu.sync_copy(deg_v, out_hbm.at[w])


@functools.partial(
    pl.kernel,
    out_type=jax.ShapeDtypeStruct((NC, NPAD, D), jnp.float32),
    mesh=_sc_mesh,
    compiler_params=_sc_params,
    scratch_types=[
        pltpu.VMEM((K,), jnp.int32),
        pltpu.VMEM((K,), jnp.int32),
        pltpu.VMEM((K, D), jnp.float32),
        pltpu.VMEM((CH, D), jnp.float32),
        pltpu.VMEM_SHARED((NPAD, D), jnp.float32),
        pltpu.SemaphoreType.DMA,
    ],
)
def _sc_propagate(h_hbm, src_hbm, dst_hbm, out_hbm,
                  sidx, didx, rows, zblk, acc_sh, sem):
    c = lax.axis_index("c")
    s = lax.axis_index("s")

    @pl.loop(0, CH)
    def _zero(i):
        for j in range(D // 16):
            zblk[i, pl.ds(j * 16, 16)] = jnp.zeros((16,), jnp.float32)

    @pl.loop(0, RPT // CH)
    def _zcopy(b):
        pltpu.sync_copy(zblk, acc_sh.at[pl.ds(s * RPT + b * CH, CH)])

    plsc.subcore_barrier()

    base = (c * NS + s) * EPW

    @pl.loop(0, NCHUNK)
    def _chunk(i):
        off = base + i * K
        pltpu.sync_copy(src_hbm.at[pl.ds(off, K)], sidx)
        pltpu.sync_copy(dst_hbm.at[pl.ds(off, K)], didx)
        pltpu.async_copy(h_hbm.at[sidx], rows, sem).wait()
        pltpu.sync_copy(rows, acc_sh.at[didx], add=True)

    plsc.subcore_barrier()

    @pl.loop(0, RPT // CH)
    def _wb(b):
        pltpu.sync_copy(acc_sh.at[pl.ds(s * RPT + b * CH, CH)], zblk)
        pltpu.sync_copy(zblk, out_hbm.at[c, pl.ds(s * RPT + b * CH, CH)])


def _dinv_of(dp_ref):
    deg = jnp.sum(dp_ref[...], axis=0) + 1.0
    return lax.rsqrt(deg)[:, None]


def _tc1_body(x_ref, w_ref, dp_ref, h_ref):
    h_ref[...] = jnp.dot(x_ref[...], w_ref[...],
                         preferred_element_type=jnp.float32) * _dinv_of(dp_ref)


def _tc2_body(ap_ref, hp_ref, dp_ref, b_ref, w_ref, o_ref):
    dinv = _dinv_of(dp_ref)
    h = ap_ref[0] + ap_ref[1] + hp_ref[...]
    h = jnp.maximum(h * dinv + b_ref[...], 0.0)
    o_ref[...] = jnp.dot(h, w_ref[...],
                         preferred_element_type=jnp.float32) * dinv


def _tc3_body(ap_ref, hp_ref, dp_ref, b_ref, batch_ref, linw_ref, linb_ref,
              o_ref, pool_acc, cnt_acc):
    i = pl.program_id(0)
    dinv = _dinv_of(dp_ref)
    h = ap_ref[0] + ap_ref[1] + hp_ref[...]
    h = jnp.maximum(h * dinv + b_ref[...], 0.0)
    bat = batch_ref[0, 0, :]
    gids = lax.broadcasted_iota(jnp.int32, (G, R), 0)
    onehot = (bat[None, :] == gids).astype(jnp.float32)
    psum = jnp.dot(onehot, h, preferred_element_type=jnp.float32)
    csum = jnp.broadcast_to(jnp.sum(onehot, axis=1, keepdims=True), (G, D))

    @pl.when(i == 0)
    def _():
        pool_acc[...] = psum
        cnt_acc[...] = csum

    @pl.when(i > 0)
    def _():
        pool_acc[...] += psum
        cnt_acc[...] += csum

    @pl.when(i == pl.num_programs(0) - 1)
    def _():
        pooled = pool_acc[...] / jnp.maximum(cnt_acc[...], 1.0)
        o_ref[...] = jnp.dot(pooled, linw_ref[...],
                             preferred_element_type=jnp.float32) + linb_ref[...]


_row_spec = pl.BlockSpec((R, D), lambda i: (i, 0))
_w_spec = pl.BlockSpec((D, D), lambda i: (0, 0))
_dp_spec = pl.BlockSpec((NW, R), lambda i: (0, i))
_ap_spec = pl.BlockSpec((2, R, D), lambda i: (0, i, 0))
_b_spec = pl.BlockSpec((1, D), lambda i: (0, 0))

_tc1 = pl.pallas_call(
    _tc1_body,
    grid=(NPAD // R,),
    in_specs=[_row_spec, _w_spec, _dp_spec],
    out_specs=_row_spec,
    out_shape=jax.ShapeDtypeStruct((NPAD, D), jnp.float32),
)

_tc2 = pl.pallas_call(
    _tc2_body,
    grid=(NPAD // R,),
    in_specs=[_ap_spec, _row_spec, _dp_spec, _b_spec, _w_spec],
    out_specs=_row_spec,
    out_shape=jax.ShapeDtypeStruct((NPAD, D), jnp.float32),
)

_tc3 = pl.pallas_call(
    _tc3_body,
    grid=(NPAD // R,),
    in_specs=[
        _ap_spec, _row_spec, _dp_spec, _b_spec,
        pl.BlockSpec((1, 1, R), lambda i: (i, 0, 0)),
        pl.BlockSpec((D, 1), lambda i: (0, 0)),
        pl.BlockSpec((1, 1), lambda i: (0, 0)),
    ],
    out_specs=pl.BlockSpec((G, 1), lambda i: (0, 0)),
    out_shape=jax.ShapeDtypeStruct((G, 1), jnp.float32),
    scratch_shapes=[
        pltpu.VMEM((G, D), jnp.float32),
        pltpu.VMEM((G, D), jnp.float32),
    ],
)


def kernel(x, edge_index, batch, W1, b1, W2, b2, lin_W, lin_b):
    src = edge_index[0]
    dst = edge_index[1]
    x_pad = jnp.pad(x, ((0, NPAD - N), (0, 0)))
    batch3 = jnp.pad(batch, (0, NPAD - N), constant_values=G).reshape(
        NPAD // R, 1, R)

    deg_parts = _sc_degree(dst)
    h1p = _tc1(x_pad, W1, deg_parts)
    acc1 = _sc_propagate(h1p, src, dst)
    h2p = _tc2(acc1, h1p, deg_parts, b1.reshape(1, D), W2)
    acc2 = _sc_propagate(h2p, src, dst)
    return _tc3(acc2, h2p, deg_parts, b2.reshape(1, D), batch3,
                lin_W, lin_b.reshape(1, 1))


# speedup vs baseline: 33.5143x; 2.4507x over previous
import functools

import jax
import jax.numpy as jnp
from jax import lax
from jax.experimental import pallas as pl
from jax.experimental.pallas import tpu as pltpu
from jax.experimental.pallas import tpu_sc as plsc

N = 10000
E = 320000
D = 128
G = 64
NC, NS = 2, 16
NW = NC * NS
NPAD = 10240
EPW = E // NW
K = 80
NCHUNK = EPW // K
NSLOT = NCHUNK + (NCHUNK % 2)
RPT = NPAD // NS
CH = K
R = 2048
DW = 16

_sc_mesh = plsc.VectorSubcoreMesh(
    core_axis_name="c", subcore_axis_name="s", num_cores=NC, num_subcores=NS)

_sc_params = pltpu.CompilerParams(
    use_tc_tiling_on_sc=False, needs_layout_passes=False)


@functools.partial(
    pl.kernel,
    out_type=jax.ShapeDtypeStruct((NW, NPAD), jnp.float32),
    mesh=_sc_mesh,
    compiler_params=_sc_params,
    scratch_types=[
        pltpu.VMEM((NCHUNK, K), jnp.int32),
        pltpu.VMEM((NPAD,), jnp.float32),
        pltpu.SemaphoreType.DMA,
    ],
)
def _sc_degree(dst_hbm, out_hbm, didx, deg_v, sem):
    c = lax.axis_index("c")
    s = lax.axis_index("s")
    w = c * NS + s

    pltpu.async_copy(dst_hbm.at[w], didx, sem)

    @pl.loop(0, NPAD // 16)
    def _zero(i):
        deg_v[pl.ds(i * 16, 16)] = jnp.zeros((16,), jnp.float32)

    pltpu.make_async_copy(dst_hbm.at[w], didx, sem).wait()
    ones16 = jnp.ones((16,), jnp.float32)

    @pl.loop(0, NCHUNK)
    def _chunk(i):
        for j in range(K // 16):
            idx = didx[i, pl.ds(j * 16, 16)]
            plsc.addupdate_scatter(deg_v, [idx], ones16)

    pltpu.sync_copy(deg_v, out_hbm.at[w])


@functools.partial(
    pl.kernel,
    out_type=jax.ShapeDtypeStruct((NC, NPAD, D), jnp.float32),
    mesh=_sc_mesh,
    compiler_params=_sc_params,
    scratch_types=[
        pltpu.VMEM((NCHUNK, K), jnp.int32),
        pltpu.VMEM((NCHUNK, K), jnp.int32),
        pltpu.VMEM((K, D), jnp.float32),
        pltpu.VMEM((K, D), jnp.float32),
        pltpu.VMEM_SHARED((NPAD, D), jnp.float32),
        pltpu.SemaphoreType.DMA,
        pltpu.SemaphoreType.DMA,
        pltpu.SemaphoreType.DMA,
        pltpu.SemaphoreType.DMA,
    ],
)
def _sc_propagate(h_hbm, src_hbm, dst_hbm, out_hbm,
                  sidx, didx, rows0, rows1, acc_sh,
                  gsem0, gsem1, ssem0, ssem1):
    c = lax.axis_index("c")
    s = lax.axis_index("s")
    w = c * NS + s

    pltpu.async_copy(src_hbm.at[w], sidx, gsem0)
    pltpu.async_copy(dst_hbm.at[w], didx, gsem1)

    @pl.loop(0, CH)
    def _zero(i):
        for j in range(D // 16):
            rows0[i, pl.ds(j * 16, 16)] = jnp.zeros((16,), jnp.float32)

    @pl.loop(0, RPT // CH)
    def _zcopy(b):
        pltpu.sync_copy(rows0, acc_sh.at[pl.ds(s * RPT + b * CH, CH)])

    pltpu.make_async_copy(src_hbm.at[w], sidx, gsem0).wait()
    pltpu.make_async_copy(dst_hbm.at[w], didx, gsem1).wait()
    plsc.subcore_barrier()

    def g_start(slot, rows, gsem):
        ch = jnp.minimum(slot, NCHUNK - 1)
        pltpu.async_copy(h_hbm.at[sidx.at[ch]], rows, gsem)

    def g_wait(rows, gsem):
        pltpu.make_async_copy(h_hbm.at[sidx.at[0]], rows, gsem).wait()

    def s_start(slot, rows, ssem):
        pltpu.async_copy(rows, acc_sh.at[didx.at[slot]], ssem, add=True)

    def s_wait(rows, ssem):
        pltpu.make_async_copy(rows, acc_sh.at[didx.at[0]], ssem).wait()

    g_start(0, rows0, gsem0)

    @pl.loop(0, NSLOT // 2)
    def _pair(g):
        s0 = 2 * g
        s1 = s0 + 1

        @pl.when(g > 0)
        def _():
            s_wait(rows1, ssem1)

        g_start(s1, rows1, gsem1)
        g_wait(rows0, gsem0)
        s_start(s0, rows0, ssem0)
        s_wait(rows0, ssem0)

        @pl.when(g < NSLOT // 2 - 1)
        def _():
            g_start(s0 + 2, rows0, gsem0)

        g_wait(rows1, gsem1)

        @pl.when(s1 < NCHUNK)
        def _():
            s_start(s1, rows1, ssem1)

    if NCHUNK % 2 == 0:
        s_wait(rows1, ssem1)

    plsc.subcore_barrier()

    @pl.loop(0, RPT // CH)
    def _wb(b):
        pltpu.sync_copy(acc_sh.at[pl.ds(s * RPT + b * CH, CH)], rows0)
        pltpu.sync_copy(rows0, out_hbm.at[c, pl.ds(s * RPT + b * CH, CH)])


def _dinv_of(dp_ref):
    deg = jnp.sum(dp_ref[...], axis=0) + 1.0
    return lax.rsqrt(deg)[:, None]


def _tc1_body(x_ref, w_ref, dp_ref, h_ref):
    h_ref[...] = jnp.dot(x_ref[...], w_ref[...],
                         preferred_element_type=jnp.float32) * _dinv_of(dp_ref)


def _tc2_body(ap_ref, hp_ref, dp_ref, b_ref, w_ref, o_ref):
    dinv = _dinv_of(dp_ref)
    h = ap_ref[0] + ap_ref[1] + hp_ref[...]
    h = jnp.maximum(h * dinv + b_ref[...], 0.0)
    o_ref[...] = jnp.dot(h, w_ref[...],
                         preferred_element_type=jnp.float32) * dinv


def _tc3_body(ap_ref, hp_ref, dp_ref, b_ref, batch_ref, linw_ref, linb_ref,
              o_ref, pool_acc, cnt_acc):
    i = pl.program_id(0)
    dinv = _dinv_of(dp_ref)
    h = ap_ref[0] + ap_ref[1] + hp_ref[...]
    h = jnp.maximum(h * dinv + b_ref[...], 0.0)
    bat = batch_ref[0, 0, :]
    gids = lax.broadcasted_iota(jnp.int32, (G, R), 0)
    onehot = (bat[None, :] == gids).astype(jnp.float32)
    psum = jnp.dot(onehot, h, preferred_element_type=jnp.float32)
    csum = jnp.broadcast_to(jnp.sum(onehot, axis=1, keepdims=True), (G, D))

    @pl.when(i == 0)
    def _():
        pool_acc[...] = psum
        cnt_acc[...] = csum

    @pl.when(i > 0)
    def _():
        pool_acc[...] += psum
        cnt_acc[...] += csum

    @pl.when(i == pl.num_programs(0) - 1)
    def _():
        pooled = pool_acc[...] / jnp.maximum(cnt_acc[...], 1.0)
        o_ref[...] = jnp.dot(pooled, linw_ref[...],
                             preferred_element_type=jnp.float32) + linb_ref[...]


_row_spec = pl.BlockSpec((R, D), lambda i: (i, 0))
_w_spec = pl.BlockSpec((D, D), lambda i: (0, 0))
_dp_spec = pl.BlockSpec((NW, R), lambda i: (0, i))
_ap_spec = pl.BlockSpec((2, R, D), lambda i: (0, i, 0))
_b_spec = pl.BlockSpec((1, D), lambda i: (0, 0))

_tc1 = pl.pallas_call(
    _tc1_body,
    grid=(NPAD // R,),
    in_specs=[_row_spec, _w_spec, _dp_spec],
    out_specs=_row_spec,
    out_shape=jax.ShapeDtypeStruct((NPAD, D), jnp.float32),
)

_tc2 = pl.pallas_call(
    _tc2_body,
    grid=(NPAD // R,),
    in_specs=[_ap_spec, _row_spec, _dp_spec, _b_spec, _w_spec],
    out_specs=_row_spec,
    out_shape=jax.ShapeDtypeStruct((NPAD, D), jnp.float32),
)

_tc3 = pl.pallas_call(
    _tc3_body,
    grid=(NPAD // R,),
    in_specs=[
        _ap_spec, _row_spec, _dp_spec, _b_spec,
        pl.BlockSpec((1, 1, R), lambda i: (i, 0, 0)),
        pl.BlockSpec((D, 1), lambda i: (0, 0)),
        pl.BlockSpec((1, 1), lambda i: (0, 0)),
    ],
    out_specs=pl.BlockSpec((G, 1), lambda i: (0, 0)),
    out_shape=jax.ShapeDtypeStruct((G, 1), jnp.float32),
    scratch_shapes=[
        pltpu.VMEM((G, D), jnp.float32),
        pltpu.VMEM((G, D), jnp.float32),
    ],
)


def kernel(x, edge_index, batch, W1, b1, W2, b2, lin_W, lin_b):
    src = edge_index[0].reshape(NW, NCHUNK, K)
    dst = edge_index[1].reshape(NW, NCHUNK, K)
    x_pad = jnp.pad(x, ((0, NPAD - N), (0, 0)))
    batch3 = jnp.pad(batch, (0, NPAD - N), constant_values=G).reshape(
        NPAD // R, 1, R)

    deg_parts = _sc_degree(dst)
    h1p = _tc1(x_pad, W1, deg_parts)
    acc1 = _sc_propagate(h1p, src, dst)
    h2p = _tc2(acc1, h1p, deg_parts, b1.reshape(1, D), W2)
    acc2 = _sc_propagate(h2p, src, dst)
    return _tc3(acc2, h2p, deg_parts, b2.reshape(1, D), batch3,
                lin_W, lin_b.reshape(1, 1))


# speedup vs baseline: 34.0806x; 1.0169x over previous
import functools

import jax
import jax.numpy as jnp
from jax import lax
from jax.experimental import pallas as pl
from jax.experimental.pallas import tpu as pltpu
from jax.experimental.pallas import tpu_sc as plsc

N = 10000
E = 320000
D = 128
G = 64
NC, NS = 2, 16
NW = NC * NS
NPAD = 10240
EPW = E // NW
K = 80
NCHUNK = EPW // K
NSLOT = NCHUNK + (NCHUNK % 2)
RPT = NPAD // NS
CH = K
R = 2048
DW = 16

_sc_mesh = plsc.VectorSubcoreMesh(
    core_axis_name="c", subcore_axis_name="s", num_cores=NC, num_subcores=NS)

_sc_params = pltpu.CompilerParams(
    use_tc_tiling_on_sc=False, needs_layout_passes=False)


@functools.partial(
    pl.kernel,
    out_type=jax.ShapeDtypeStruct((NW, NPAD), jnp.float32),
    mesh=_sc_mesh,
    compiler_params=_sc_params,
    scratch_types=[
        pltpu.VMEM((NCHUNK, K), jnp.int32),
        pltpu.VMEM((NPAD,), jnp.float32),
        pltpu.SemaphoreType.DMA,
    ],
)
def _sc_degree(dst_hbm, out_hbm, didx, deg_v, sem):
    c = lax.axis_index("c")
    s = lax.axis_index("s")
    w = c * NS + s

    pltpu.async_copy(dst_hbm.at[w], didx, sem)

    @pl.loop(0, NPAD // 16)
    def _zero(i):
        deg_v[pl.ds(i * 16, 16)] = jnp.zeros((16,), jnp.float32)

    pltpu.make_async_copy(dst_hbm.at[w], didx, sem).wait()
    ones16 = jnp.ones((16,), jnp.float32)

    @pl.loop(0, NCHUNK)
    def _chunk(i):
        for j in range(K // 16):
            idx = didx[i, pl.ds(j * 16, 16)]
            plsc.addupdate_scatter(deg_v, [idx], ones16)

    pltpu.sync_copy(deg_v, out_hbm.at[w])


@functools.partial(
    pl.kernel,
    out_type=jax.ShapeDtypeStruct((NC, NPAD, D), jnp.float32),
    mesh=_sc_mesh,
    compiler_params=_sc_params,
    scratch_types=[
        pltpu.VMEM((NCHUNK, K), jnp.int32),
        pltpu.VMEM((NCHUNK, K), jnp.int32),
        pltpu.VMEM((K, D), jnp.float32),
        pltpu.VMEM((K, D), jnp.float32),
        pltpu.VMEM_SHARED((NPAD, D), jnp.float32),
        pltpu.SemaphoreType.DMA,
        pltpu.SemaphoreType.DMA,
        pltpu.SemaphoreType.DMA,
        pltpu.SemaphoreType.DMA,
    ],
)
def _sc_propagate(h_hbm, src_hbm, dst_hbm, out_hbm,
                  sidx, didx, rows0, rows1, acc_sh,
                  gsem0, gsem1, ssem0, ssem1):
    c = lax.axis_index("c")
    s = lax.axis_index("s")
    w = c * NS + s

    pltpu.async_copy(src_hbm.at[w], sidx, gsem0)
    pltpu.async_copy(dst_hbm.at[w], didx, gsem1)

    @pl.loop(0, CH)
    def _zero(i):
        for j in range(D // 16):
            rows0[i, pl.ds(j * 16, 16)] = jnp.zeros((16,), jnp.float32)

    for b in range(RPT // CH):
        pltpu.async_copy(rows0, acc_sh.at[pl.ds(s * RPT + b * CH, CH)], ssem0)
    for b in range(RPT // CH):
        pltpu.make_async_copy(rows0, acc_sh.at[pl.ds(s * RPT, CH)], ssem0).wait()

    pltpu.make_async_copy(src_hbm.at[w], sidx, gsem0).wait()
    pltpu.make_async_copy(dst_hbm.at[w], didx, gsem1).wait()
    plsc.subcore_barrier()

    def g_start(slot, rows, gsem):
        ch = jnp.minimum(slot, NCHUNK - 1)
        pltpu.async_copy(h_hbm.at[sidx.at[ch]], rows, gsem)

    def g_wait(rows, gsem):
        pltpu.make_async_copy(h_hbm.at[sidx.at[0]], rows, gsem).wait()

    def s_start(slot, rows, ssem):
        pltpu.async_copy(rows, acc_sh.at[didx.at[slot]], ssem, add=True)

    def s_wait(rows, ssem):
        pltpu.make_async_copy(rows, acc_sh.at[didx.at[0]], ssem).wait()

    g_start(0, rows0, gsem0)

    @pl.loop(0, NSLOT // 2)
    def _pair(g):
        s0 = 2 * g
        s1 = s0 + 1

        @pl.when(g > 0)
        def _():
            s_wait(rows1, ssem1)

        g_start(s1, rows1, gsem1)
        g_wait(rows0, gsem0)
        s_start(s0, rows0, ssem0)
        s_wait(rows0, ssem0)

        @pl.when(g < NSLOT // 2 - 1)
        def _():
            g_start(s0 + 2, rows0, gsem0)

        g_wait(rows1, gsem1)

        @pl.when(s1 < NCHUNK)
        def _():
            s_start(s1, rows1, ssem1)

    if NCHUNK % 2 == 0:
        s_wait(rows1, ssem1)

    plsc.subcore_barrier()

    bufs = (rows0, rows1)
    isems = (gsem0, gsem1)
    osems = (ssem0, ssem1)
    NB = RPT // CH

    def _wb_row(b):
        return pl.ds(s * RPT + b * CH, CH)

    for b in range(NB + 1):
        if b < NB:
            if b >= 2:
                pltpu.make_async_copy(
                    bufs[b % 2], out_hbm.at[c, _wb_row(0)], osems[b % 2]).wait()
            pltpu.async_copy(acc_sh.at[_wb_row(b)], bufs[b % 2], isems[b % 2])
        if b >= 1:
            pltpu.make_async_copy(
                acc_sh.at[_wb_row(0)], bufs[(b - 1) % 2], isems[(b - 1) % 2]).wait()
            pltpu.async_copy(
                bufs[(b - 1) % 2], out_hbm.at[c, _wb_row(b - 1)], osems[(b - 1) % 2])
    pltpu.make_async_copy(bufs[0], out_hbm.at[c, _wb_row(0)], osems[0]).wait()
    pltpu.make_async_copy(bufs[1], out_hbm.at[c, _wb_row(0)], osems[1]).wait()


def _dinv_of(dp_ref):
    deg = jnp.sum(dp_ref[...], axis=0) + 1.0
    return lax.rsqrt(deg)[:, None]


def _tc1_body(x_ref, w_ref, dp_ref, h_ref):
    h_ref[...] = jnp.dot(x_ref[...], w_ref[...],
                         preferred_element_type=jnp.float32) * _dinv_of(dp_ref)


def _tc2_body(ap_ref, hp_ref, dp_ref, b_ref, w_ref, o_ref):
    dinv = _dinv_of(dp_ref)
    h = ap_ref[0] + ap_ref[1] + hp_ref[...]
    h = jnp.maximum(h * dinv + b_ref[...], 0.0)
    o_ref[...] = jnp.dot(h, w_ref[...],
                         preferred_element_type=jnp.float32) * dinv


def _tc3_body(ap_ref, hp_ref, dp_ref, b_ref, batch_ref, linw_ref, linb_ref,
              o_ref, pool_acc, cnt_acc):
    i = pl.program_id(0)
    dinv = _dinv_of(dp_ref)
    h = ap_ref[0] + ap_ref[1] + hp_ref[...]
    h = jnp.maximum(h * dinv + b_ref[...], 0.0)
    bat = batch_ref[0, 0, :]
    gids = lax.broadcasted_iota(jnp.int32, (G, R), 0)
    onehot = (bat[None, :] == gids).astype(jnp.float32)
    psum = jnp.dot(onehot, h, preferred_element_type=jnp.float32)
    csum = jnp.broadcast_to(jnp.sum(onehot, axis=1, keepdims=True), (G, D))

    @pl.when(i == 0)
    def _():
        pool_acc[...] = psum
        cnt_acc[...] = csum

    @pl.when(i > 0)
    def _():
        pool_acc[...] += psum
        cnt_acc[...] += csum

    @pl.when(i == pl.num_programs(0) - 1)
    def _():
        pooled = pool_acc[...] / jnp.maximum(cnt_acc[...], 1.0)
        o_ref[...] = jnp.dot(pooled, linw_ref[...],
                             preferred_element_type=jnp.float32) + linb_ref[...]


_row_spec = pl.BlockSpec((R, D), lambda i: (i, 0))
_w_spec = pl.BlockSpec((D, D), lambda i: (0, 0))
_dp_spec = pl.BlockSpec((NW, R), lambda i: (0, i))
_ap_spec = pl.BlockSpec((2, R, D), lambda i: (0, i, 0))
_b_spec = pl.BlockSpec((1, D), lambda i: (0, 0))

_tc1 = pl.pallas_call(
    _tc1_body,
    grid=(NPAD // R,),
    in_specs=[_row_spec, _w_spec, _dp_spec],
    out_specs=_row_spec,
    out_shape=jax.ShapeDtypeStruct((NPAD, D), jnp.float32),
)

_tc2 = pl.pallas_call(
    _tc2_body,
    grid=(NPAD // R,),
    in_specs=[_ap_spec, _row_spec, _dp_spec, _b_spec, _w_spec],
    out_specs=_row_spec,
    out_shape=jax.ShapeDtypeStruct((NPAD, D), jnp.float32),
)

_tc3 = pl.pallas_call(
    _tc3_body,
    grid=(NPAD // R,),
    in_specs=[
        _ap_spec, _row_spec, _dp_spec, _b_spec,
        pl.BlockSpec((1, 1, R), lambda i: (i, 0, 0)),
        pl.BlockSpec((D, 1), lambda i: (0, 0)),
        pl.BlockSpec((1, 1), lambda i: (0, 0)),
    ],
    out_specs=pl.BlockSpec((G, 1), lambda i: (0, 0)),
    out_shape=jax.ShapeDtypeStruct((G, 1), jnp.float32),
    scratch_shapes=[
        pltpu.VMEM((G, D), jnp.float32),
        pltpu.VMEM((G, D), jnp.float32),
    ],
)


def kernel(x, edge_index, batch, W1, b1, W2, b2, lin_W, lin_b):
    src = edge_index[0].reshape(NW, NCHUNK, K)
    dst = edge_index[1].reshape(NW, NCHUNK, K)
    x_pad = jnp.pad(x, ((0, NPAD - N), (0, 0)))
    batch3 = jnp.pad(batch, (0, NPAD - N), constant_values=G).reshape(
        NPAD // R, 1, R)

    deg_parts = _sc_degree(dst)
    h1p = _tc1(x_pad, W1, deg_parts)
    acc1 = _sc_propagate(h1p, src, dst)
    h2p = _tc2(acc1, h1p, deg_parts, b1.reshape(1, D), W2)
    acc2 = _sc_propagate(h2p, src, dst)
    return _tc3(acc2, h2p, deg_parts, b2.reshape(1, D), batch3,
                lin_W, lin_b.reshape(1, 1))


# speedup vs baseline: 34.0864x; 1.0002x over previous
import functools

import jax
import jax.numpy as jnp
from jax import lax
from jax.experimental import pallas as pl
from jax.experimental.pallas import tpu as pltpu
from jax.experimental.pallas import tpu_sc as plsc

N = 10000
E = 320000
D = 128
G = 64
NC, NS = 2, 16
NW = NC * NS
NPAD = 10240
EPW = E // NW
K = 80
NCHUNK = EPW // K
NSLOT = 3 * ((NCHUNK + 2) // 3)
RPT = NPAD // NS
CH = K
NBF = RPT // CH
TAIL = RPT - NBF * CH
R = 2048
DW = 16

_sc_mesh = plsc.VectorSubcoreMesh(
    core_axis_name="c", subcore_axis_name="s", num_cores=NC, num_subcores=NS)

_sc_params = pltpu.CompilerParams(
    use_tc_tiling_on_sc=False, needs_layout_passes=False)


@functools.partial(
    pl.kernel,
    out_type=jax.ShapeDtypeStruct((NW, NPAD), jnp.float32),
    mesh=_sc_mesh,
    compiler_params=_sc_params,
    scratch_types=[
        pltpu.VMEM((NCHUNK, K), jnp.int32),
        pltpu.VMEM((NPAD,), jnp.float32),
        pltpu.SemaphoreType.DMA,
    ],
)
def _sc_degree(dst_hbm, out_hbm, didx, deg_v, sem):
    c = lax.axis_index("c")
    s = lax.axis_index("s")
    w = c * NS + s

    pltpu.async_copy(dst_hbm.at[w], didx, sem)

    @pl.loop(0, NPAD // 16)
    def _zero(i):
        deg_v[pl.ds(i * 16, 16)] = jnp.zeros((16,), jnp.float32)

    pltpu.make_async_copy(dst_hbm.at[w], didx, sem).wait()
    ones16 = jnp.ones((16,), jnp.float32)

    @pl.loop(0, NCHUNK)
    def _chunk(i):
        for j in range(K // 16):
            idx = didx[i, pl.ds(j * 16, 16)]
            plsc.addupdate_scatter(deg_v, [idx], ones16)

    pltpu.sync_copy(deg_v, out_hbm.at[w])


@functools.partial(
    pl.kernel,
    out_type=jax.ShapeDtypeStruct((NC, NPAD, D), jnp.float32),
    mesh=_sc_mesh,
    compiler_params=_sc_params,
    scratch_types=[
        pltpu.VMEM((NCHUNK, K), jnp.int32),
        pltpu.VMEM((NCHUNK, K), jnp.int32),
        pltpu.VMEM((K, D), jnp.float32),
        pltpu.VMEM((K, D), jnp.float32),
        pltpu.VMEM_SHARED((NPAD, D), jnp.float32),
        pltpu.SemaphoreType.DMA,
        pltpu.SemaphoreType.DMA,
        pltpu.SemaphoreType.DMA,
        pltpu.SemaphoreType.DMA,
    ],
)
def _sc_propagate(h_hbm, src_hbm, dst_hbm, out_hbm,
                  sidx, didx, rows0, rows1, acc_sh,
                  gsem0, gsem1, ssem0, ssem1):
    c = lax.axis_index("c")
    s = lax.axis_index("s")
    w = c * NS + s

    pltpu.async_copy(src_hbm.at[w], sidx, gsem0)
    pltpu.async_copy(dst_hbm.at[w], didx, gsem1)

    @pl.loop(0, CH)
    def _zero(i):
        for j in range(D // 16):
            rows0[i, pl.ds(j * 16, 16)] = jnp.zeros((16,), jnp.float32)

    for b in range(NBF):
        pltpu.async_copy(rows0, acc_sh.at[pl.ds(s * RPT + b * CH, CH)], ssem0)
    if TAIL:
        pltpu.async_copy(rows0.at[pl.ds(0, TAIL)],
                         acc_sh.at[pl.ds(s * RPT + NBF * CH, TAIL)], ssem1)
    for b in range(NBF):
        pltpu.make_async_copy(rows0, acc_sh.at[pl.ds(s * RPT, CH)], ssem0).wait()
    if TAIL:
        pltpu.make_async_copy(rows0.at[pl.ds(0, TAIL)],
                              acc_sh.at[pl.ds(s * RPT, TAIL)], ssem1).wait()

    pltpu.make_async_copy(src_hbm.at[w], sidx, gsem0).wait()
    pltpu.make_async_copy(dst_hbm.at[w], didx, gsem1).wait()
    plsc.subcore_barrier()

    def g_start(slot, rows, gsem):
        ch = jnp.minimum(slot, NCHUNK - 1)
        pltpu.async_copy(h_hbm.at[sidx.at[ch]], rows, gsem)

    def g_wait(rows, gsem):
        pltpu.make_async_copy(h_hbm.at[sidx.at[0]], rows, gsem).wait()

    def s_start(slot, rows, ssem):
        pltpu.async_copy(rows, acc_sh.at[didx.at[slot]], ssem, add=True)

    def s_wait(rows, ssem):
        pltpu.make_async_copy(rows, acc_sh.at[didx.at[0]], ssem).wait()

    g_start(0, rows0, gsem0)

    @pl.loop(0, NSLOT // 2)
    def _pair(g):
        s0 = 2 * g
        s1 = s0 + 1

        @pl.when(g > 0)
        def _():
            s_wait(rows1, ssem1)

        g_start(s1, rows1, gsem1)
        g_wait(rows0, gsem0)
        s_start(s0, rows0, ssem0)
        s_wait(rows0, ssem0)

        @pl.when(g < NSLOT // 2 - 1)
        def _():
            g_start(s0 + 2, rows0, gsem0)

        g_wait(rows1, gsem1)

        @pl.when(s1 < NCHUNK)
        def _():
            s_start(s1, rows1, ssem1)

    if NCHUNK % 2 == 0:
        s_wait(rows1, ssem1)

    plsc.subcore_barrier()

    bufs = (rows0, rows1)
    isems = (gsem0, gsem1)
    osems = (ssem0, ssem1)
    NBLK = NBF + (1 if TAIL else 0)

    def _nr(b):
        return CH if b < NBF else TAIL

    for b in range(NBLK):
        buf, isem, osem = bufs[b % 2], isems[b % 2], osems[b % 2]
        nr = _nr(b)
        if b >= 2:
            pltpu.make_async_copy(
                buf.at[pl.ds(0, CH)],
                out_hbm.at[c, pl.ds(s * RPT, CH)], osem).wait()
        pltpu.async_copy(
            acc_sh.at[pl.ds(s * RPT + b * CH, nr)], buf.at[pl.ds(0, nr)], isem)
        pltpu.make_async_copy(
            acc_sh.at[pl.ds(s * RPT, nr)], buf.at[pl.ds(0, nr)], isem).wait()
        pltpu.async_copy(
            buf.at[pl.ds(0, nr)], out_hbm.at[c, pl.ds(s * RPT + b * CH, nr)],
            osem)
    for b in (NBLK - 2, NBLK - 1):
        nr = _nr(b)
        pltpu.make_async_copy(
            bufs[b % 2].at[pl.ds(0, nr)],
            out_hbm.at[c, pl.ds(s * RPT, nr)], osems[b % 2]).wait()


def _dinv_of(dp_ref):
    deg = jnp.sum(dp_ref[...], axis=0) + 1.0
    return lax.rsqrt(deg)[:, None]


def _tc1_body(x_ref, w_ref, dp_ref, h_ref):
    h_ref[...] = jnp.dot(x_ref[...], w_ref[...],
                         preferred_element_type=jnp.float32) * _dinv_of(dp_ref)


def _tc2_body(ap_ref, hp_ref, dp_ref, b_ref, w_ref, o_ref):
    dinv = _dinv_of(dp_ref)
    h = ap_ref[0] + ap_ref[1] + hp_ref[...]
    h = jnp.maximum(h * dinv + b_ref[...], 0.0)
    o_ref[...] = jnp.dot(h, w_ref[...],
                         preferred_element_type=jnp.float32) * dinv


def _tc3_body(ap_ref, hp_ref, dp_ref, b_ref, batch_ref, linw_ref, linb_ref,
              o_ref, pool_acc, cnt_acc):
    i = pl.program_id(0)
    dinv = _dinv_of(dp_ref)
    h = ap_ref[0] + ap_ref[1] + hp_ref[...]
    h = jnp.maximum(h * dinv + b_ref[...], 0.0)
    bat = batch_ref[0, 0, :]
    gids = lax.broadcasted_iota(jnp.int32, (G, R), 0)
    onehot = (bat[None, :] == gids).astype(jnp.float32)
    psum = jnp.dot(onehot, h, preferred_element_type=jnp.float32)
    csum = jnp.broadcast_to(jnp.sum(onehot, axis=1, keepdims=True), (G, D))

    @pl.when(i == 0)
    def _():
        pool_acc[...] = psum
        cnt_acc[...] = csum

    @pl.when(i > 0)
    def _():
        pool_acc[...] += psum
        cnt_acc[...] += csum

    @pl.when(i == pl.num_programs(0) - 1)
    def _():
        pooled = pool_acc[...] / jnp.maximum(cnt_acc[...], 1.0)
        o_ref[...] = jnp.dot(pooled, linw_ref[...],
                             preferred_element_type=jnp.float32) + linb_ref[...]


_row_spec = pl.BlockSpec((R, D), lambda i: (i, 0))
_w_spec = pl.BlockSpec((D, D), lambda i: (0, 0))
_dp_spec = pl.BlockSpec((NW, R), lambda i: (0, i))
_ap_spec = pl.BlockSpec((2, R, D), lambda i: (0, i, 0))
_b_spec = pl.BlockSpec((1, D), lambda i: (0, 0))

_tc1 = pl.pallas_call(
    _tc1_body,
    grid=(NPAD // R,),
    in_specs=[_row_spec, _w_spec, _dp_spec],
    out_specs=_row_spec,
    out_shape=jax.ShapeDtypeStruct((NPAD, D), jnp.float32),
)

_tc2 = pl.pallas_call(
    _tc2_body,
    grid=(NPAD // R,),
    in_specs=[_ap_spec, _row_spec, _dp_spec, _b_spec, _w_spec],
    out_specs=_row_spec,
    out_shape=jax.ShapeDtypeStruct((NPAD, D), jnp.float32),
)

_tc3 = pl.pallas_call(
    _tc3_body,
    grid=(NPAD // R,),
    in_specs=[
        _ap_spec, _row_spec, _dp_spec, _b_spec,
        pl.BlockSpec((1, 1, R), lambda i: (i, 0, 0)),
        pl.BlockSpec((D, 1), lambda i: (0, 0)),
        pl.BlockSpec((1, 1), lambda i: (0, 0)),
    ],
    out_specs=pl.BlockSpec((G, 1), lambda i: (0, 0)),
    out_shape=jax.ShapeDtypeStruct((G, 1), jnp.float32),
    scratch_shapes=[
        pltpu.VMEM((G, D), jnp.float32),
        pltpu.VMEM((G, D), jnp.float32),
    ],
)


def kernel(x, edge_index, batch, W1, b1, W2, b2, lin_W, lin_b):
    src = edge_index[0].reshape(NW, NCHUNK, K)
    dst = edge_index[1].reshape(NW, NCHUNK, K)
    x_pad = jnp.pad(x, ((0, NPAD - N), (0, 0)))
    batch3 = jnp.pad(batch, (0, NPAD - N), constant_values=G).reshape(
        NPAD // R, 1, R)

    deg_parts = _sc_degree(dst)
    h1p = _tc1(x_pad, W1, deg_parts)
    acc1 = _sc_propagate(h1p, src, dst)
    h2p = _tc2(acc1, h1p, deg_parts, b1.reshape(1, D), W2)
    acc2 = _sc_propagate(h2p, src, dst)
    return _tc3(acc2, h2p, deg_parts, b2.reshape(1, D), batch3,
                lin_W, lin_b.reshape(1, 1))
